# baseline (speedup 1.0000x reference)
import functools

import jax
import jax.numpy as jnp
from jax import lax
from jax.experimental import pallas as pl
from jax.experimental.pallas import tpu as pltpu
from jax.experimental.pallas import tpu_sc as plsc

_B = 4096
_F = 26
_E = 16
_BF = _B * _F
_NW = 32
_PW = _BF // _NW
_CL = 128
_NCH = _PW // _CL
_BB = 512
_H = 400
_D_IN = _F * _E
_INV_STD = 0.9999950000374997


def _sc_gather(emb, fw16, idx_r, idx_hi, idx_lo):
    mesh = plsc.VectorSubcoreMesh(core_axis_name="c", subcore_axis_name="s")

    @functools.partial(
        pl.kernel,
        mesh=mesh,
        out_type=[
            jax.ShapeDtypeStruct((_NW, _PW, _E), jnp.float32),
            jax.ShapeDtypeStruct((_NW, _PW), jnp.float32),
        ],
        scratch_types=[
            pltpu.VMEM((_NCH, _CL), jnp.int32),
            pltpu.VMEM((_NCH, _CL), jnp.int32),
            pltpu.VMEM((_NCH, _CL), jnp.int32),
            pltpu.VMEM((_PW, _E), jnp.float32),
            pltpu.VMEM((_PW, _E), jnp.float32),
            pltpu.VMEM((_PW,), jnp.float32),
            pltpu.SemaphoreType.DMA,
        ],
        compiler_params=pltpu.CompilerParams(
            use_tc_tiling_on_sc=False, needs_layout_passes=False),
    )
    def gather_kernel(emb_hbm, fw16_hbm, idx_hbm, idxhi_hbm, idxlo_hbm,
                      out_e, out_f,
                      idx_v, idxhi_v, idxlo_v, erows, frows16, fwvals, sem):
        wid = lax.axis_index("s") * 2 + lax.axis_index("c")
        pltpu.sync_copy(idx_hbm.at[wid], idx_v)
        pltpu.sync_copy(idxhi_hbm.at[wid], idxhi_v)
        pltpu.sync_copy(idxlo_hbm.at[wid], idxlo_v)

        def fire(c, carry):
            pltpu.async_copy(emb_hbm.at[idx_v.at[c]],
                             erows.at[pl.ds(c * _CL, _CL)], sem)
            pltpu.async_copy(fw16_hbm.at[idxhi_v.at[c]],
                             frows16.at[pl.ds(c * _CL, _CL)], sem)
            return carry

        lax.fori_loop(0, _NCH, fire, 0)

        def drain(c, carry):
            pltpu.make_async_copy(
                emb_hbm.at[idx_v.at[c]],
                erows.at[pl.ds(c * _CL, _CL)], sem).wait()
            pltpu.make_async_copy(
                fw16_hbm.at[idxhi_v.at[c]],
                frows16.at[pl.ds(c * _CL, _CL)], sem).wait()
            return carry

        lax.fori_loop(0, _NCH, drain, 0)
        pltpu.sync_copy(erows, out_e.at[wid])

        lane_iota = lax.iota(jnp.int32, 16)

        def select(k, carry):
            c = k // (_CL // 16)
            j = k - c * (_CL // 16)
            lane = idxlo_v[c, pl.ds(j * 16, 16)]
            rows = lane_iota + k * 16
            fwvals[pl.ds(k * 16, 16)] = plsc.load_gather(frows16, [rows, lane])
            return carry

        lax.fori_loop(0, _PW // 16, select, 0)
        pltpu.sync_copy(fwvals, out_f.at[wid])

    return gather_kernel(emb, fw16, idx_r, idx_hi, idx_lo)


def _tc_body(e_ref, fw_ref, fv_ref,
             w1_ref, b1_ref, g1_ref, be1_ref,
             w2_ref, b2_ref, g2_ref, be2_ref,
             w3_ref, b3_ref, g3_ref, be3_ref,
             wfc_ref, bfc_ref, out_ref):
    fv = fv_ref[...]
    fw = fw_ref[...]
    e_raw = e_ref[...]

    rep_f = lax.broadcasted_iota(jnp.int32, (_F, _D_IN), 0)
    rep_j = lax.broadcasted_iota(jnp.int32, (_F, _D_IN), 1) // _E
    rep = (rep_f == rep_j).astype(jnp.float32)
    fvx = jnp.dot(fv, rep, preferred_element_type=jnp.float32, precision=lax.Precision.HIGHEST)
    e = e_raw * fvx

    sum_j = lax.broadcasted_iota(jnp.int32, (_D_IN, _E), 0) % _E
    sum_k = lax.broadcasted_iota(jnp.int32, (_D_IN, _E), 1)
    smat = (sum_j == sum_k).astype(jnp.float32)
    summed = jnp.dot(e, smat, preferred_element_type=jnp.float32, precision=lax.Precision.HIGHEST)
    sumsq = jnp.dot(e * e, smat, preferred_element_type=jnp.float32, precision=lax.Precision.HIGHEST)
    y_secd = 0.5 * (summed * summed - sumsq)

    y_first = fw * fv

    def dot16(a, b):
        return jnp.dot(a.astype(jnp.bfloat16), b.astype(jnp.bfloat16),
                       preferred_element_type=jnp.float32)

    h = dot16(e, w1_ref[...]) + b1_ref[...]
    h = jnp.maximum(h * (_INV_STD * g1_ref[...]) + be1_ref[...], 0.0)
    h = dot16(h, w2_ref[...]) + b2_ref[...]
    h = jnp.maximum(h * (_INV_STD * g2_ref[...]) + be2_ref[...], 0.0)
    h = dot16(h, w3_ref[...]) + b3_ref[...]
    h = jnp.maximum(h * (_INV_STD * g3_ref[...]) + be3_ref[...], 0.0)

    wfc = wfc_ref[...]
    out = (dot16(y_first, wfc[0:_F, :])
           + dot16(y_secd, wfc[_F:_F + _E, :])
           + dot16(h, wfc[_F + _E:, :])
           + bfc_ref[...])
    out_ref[...] = out


def _tc_dense(e_raw, fw, fv, W1, b1, g1, be1, W2, b2, g2, be2,
              W3, b3, g3, be3, Wfc, bfc):
    grid = (_B // _BB,)

    def row_block(i):
        return (i, 0)

    def whole(i):
        return (0, 0)

    bspec = lambda shape, imap: pl.BlockSpec(shape, imap)
    in_specs = [
        bspec((_BB, _D_IN), row_block),
        bspec((_BB, _F), row_block),
        bspec((_BB, _F), row_block),
        bspec((_D_IN, _H), whole), bspec((1, _H), whole),
        bspec((1, _H), whole), bspec((1, _H), whole),
        bspec((_H, _H), whole), bspec((1, _H), whole),
        bspec((1, _H), whole), bspec((1, _H), whole),
        bspec((_H, _H), whole), bspec((1, _H), whole),
        bspec((1, _H), whole), bspec((1, _H), whole),
        bspec((_F + _E + _H, 1), whole), bspec((1, 1), whole),
    ]
    return pl.pallas_call(
        _tc_body,
        grid=grid,
        in_specs=in_specs,
        out_specs=pl.BlockSpec((_BB, 1), row_block),
        out_shape=jax.ShapeDtypeStruct((_B, 1), jnp.float32),
    )(e_raw, fw, fv, W1, b1, g1, be1, W2, b2, g2, be2,
      W3, b3, g3, be3, Wfc, bfc)


def kernel(feat_index, feat_value, first_w, emb,
           W1, b1, g1, be1, W2, b2, g2, be2, W3, b3, g3, be3,
           Wfc, bfc):
    idx_r = feat_index.astype(jnp.int32).reshape(_NW, _NCH, _CL)
    e_rows, f_rows = _sc_gather(emb, first_w.reshape(-1, _E), idx_r,
                                idx_r >> 4, idx_r & 15)
    e_raw = e_rows.reshape(_B, _D_IN)
    fw = f_rows.reshape(_B, _F)
    out = _tc_dense(
        e_raw, fw, feat_value,
        W1, b1.reshape(1, _H), g1.reshape(1, _H), be1.reshape(1, _H),
        W2, b2.reshape(1, _H), g2.reshape(1, _H), be2.reshape(1, _H),
        W3, b3.reshape(1, _H), g3.reshape(1, _H), be3.reshape(1, _H),
        Wfc, bfc.reshape(1, 1))
    return out
